# baseline (speedup 1.0000x reference)
import functools

import jax
import jax.numpy as jnp
from jax import lax
from jax.experimental import pallas as pl
from jax.experimental.pallas import tpu as pltpu
from jax.experimental.pallas import tpu_sc as plsc

N = 10000
E = 320000
D = 128

NC = 2
NS = 16
NW = NC * NS
CHUNK = 128
NBUF = 2
GROUP = 8
NCHUNKS = 80
NGROUPS = NCHUNKS // GROUP
EPW = NCHUNKS * CHUNK
E_PAD = NW * EPW
N_PAD = 10240
ROWS_PER_TILE = N_PAD // NS
ROW_STEPS = ROWS_PER_TILE // CHUNK

ROW_BLK = 10000
GRID = N // ROW_BLK


def _msg_body(x_ref, w_ref, b_ref, wa_ref, bn_ref, h_ref, t_ref):
    xv = x_ref[...]
    acc = jnp.dot(xv, w_ref[...], preferred_element_type=jnp.float32)
    h_ref[...] = jnp.maximum(acc + b_ref[...], 0.0)
    t_ref[...] = jnp.dot(xv, wa_ref[...],
                         preferred_element_type=jnp.float32) + bn_ref[...]


def _msg_transform(x, w, b, wa, bn):
    return pl.pallas_call(
        _msg_body,
        grid=(GRID,),
        in_specs=[
            pl.BlockSpec((ROW_BLK, D), lambda i: (i, 0)),
            pl.BlockSpec((D, D), lambda i: (0, 0)),
            pl.BlockSpec((1, D), lambda i: (0, 0)),
            pl.BlockSpec((D, D), lambda i: (0, 0)),
            pl.BlockSpec((1, D), lambda i: (0, 0)),
        ],
        out_specs=[pl.BlockSpec((ROW_BLK, D), lambda i: (i, 0)),
                   pl.BlockSpec((ROW_BLK, D), lambda i: (i, 0))],
        out_shape=[jax.ShapeDtypeStruct((N, D), jnp.float32),
                   jax.ShapeDtypeStruct((N, D), jnp.float32)],
    )(x, w, b, wa, bn)


@functools.cache
def _make_segment_sum_sc():
    mesh = plsc.VectorSubcoreMesh(
        core_axis_name="c", subcore_axis_name="s",
        num_cores=NC, num_subcores=NS)
    return pl.kernel(
        _segment_sum_body,
        out_type=jax.ShapeDtypeStruct((NC, N_PAD, D), jnp.float32),
        mesh=mesh,
        scratch_types=[
            pltpu.VMEM((2, GROUP, CHUNK), jnp.int32),
            pltpu.VMEM((2, GROUP, CHUNK), jnp.int32),
            pltpu.VMEM((NBUF, CHUNK, D), jnp.float32),
            pltpu.VMEM_SHARED((N_PAD, D), jnp.float32),
            [pltpu.SemaphoreType.DMA] * NBUF,
            [pltpu.SemaphoreType.DMA] * NBUF,
        ],
    )


def _segment_sum_body(h_hbm, src_hbm, dst_hbm, out_hbm,
                      src_v, dst_v, rows_v, acc_sh, gsem, ssem):
    c = lax.axis_index("c")
    s = lax.axis_index("s")
    wid = c * NS + s

    zero16 = jnp.zeros((16,), jnp.float32)

    def _z(i, _):
        rows_v[0, i // (D // 16), pl.ds((i % (D // 16)) * 16, 16)] = zero16
        return 0

    lax.fori_loop(0, CHUNK * (D // 16), _z, 0)

    row0 = s * ROWS_PER_TILE

    def _zc(j, _):
        pltpu.sync_copy(rows_v.at[0],
                        acc_sh.at[pl.ds(row0 + j * CHUNK, CHUNK)])
        return 0

    lax.fori_loop(0, ROW_STEPS, _zc, 0)
    plsc.subcore_barrier()

    pltpu.sync_copy(src_hbm.at[wid, pl.ds(0, GROUP)], src_v.at[0])
    pltpu.sync_copy(dst_hbm.at[wid, pl.ds(0, GROUP)], dst_v.at[0])
    for b in range(NBUF):
        pltpu.async_copy(h_hbm.at[src_v.at[0, b]], rows_v.at[b], gsem[b])

    def _outer(g, _):
        p = g % 2
        q = (g + 1) % 2

        @pl.when(g + 1 < NGROUPS)
        def _():
            pltpu.sync_copy(src_hbm.at[wid, pl.ds((g + 1) * GROUP, GROUP)],
                            src_v.at[q])
            pltpu.sync_copy(dst_hbm.at[wid, pl.ds((g + 1) * GROUP, GROUP)],
                            dst_v.at[q])

        for b in range(GROUP):
            r = b % NBUF
            pltpu.make_async_copy(
                h_hbm.at[src_v.at[p, b]], rows_v.at[r], gsem[r]).wait()
            pltpu.async_copy(
                rows_v.at[r], acc_sh.at[dst_v.at[p, b]], ssem[r], add=True)
            pltpu.make_async_copy(
                rows_v.at[r], acc_sh.at[dst_v.at[p, b]], ssem[r]).wait()
            if b + NBUF < GROUP:
                pltpu.async_copy(
                    h_hbm.at[src_v.at[p, b + NBUF]], rows_v.at[r], gsem[r])
            else:
                @pl.when(g + 1 < NGROUPS)
                def _():
                    pltpu.async_copy(
                        h_hbm.at[src_v.at[q, b + NBUF - GROUP]],
                        rows_v.at[r], gsem[r])
        return 0

    lax.fori_loop(0, NGROUPS, _outer, 0)
    plsc.subcore_barrier()

    for j in range(ROW_STEPS):
        r = j % NBUF
        rr = row0 + j * CHUNK
        if j >= NBUF:
            pltpu.make_async_copy(
                rows_v.at[r], out_hbm.at[c, pl.ds(row0, CHUNK)],
                ssem[r]).wait()
        pltpu.sync_copy(acc_sh.at[pl.ds(rr, CHUNK)], rows_v.at[r])
        pltpu.async_copy(rows_v.at[r], out_hbm.at[c, pl.ds(rr, CHUNK)],
                         ssem[r])
    for r in range(min(NBUF, ROW_STEPS)):
        pltpu.make_async_copy(
            rows_v.at[r], out_hbm.at[c, pl.ds(row0, CHUNK)], ssem[r]).wait()


def _next_body(t_ref, p0_ref, p1_ref, wb_ref, o_ref):
    pooled = p0_ref[0] + p1_ref[0]
    acc = t_ref[...] + jnp.dot(pooled, wb_ref[...],
                               preferred_element_type=jnp.float32)
    o_ref[...] = jnp.maximum(acc, 0.0)


def _next_state(t, partials, wb):
    return pl.pallas_call(
        _next_body,
        grid=(GRID,),
        in_specs=[
            pl.BlockSpec((ROW_BLK, D), lambda i: (i, 0)),
            pl.BlockSpec((1, ROW_BLK, D), lambda i: (0, i, 0)),
            pl.BlockSpec((1, ROW_BLK, D), lambda i: (1, i, 0)),
            pl.BlockSpec((D, D), lambda i: (0, 0)),
        ],
        out_specs=pl.BlockSpec((ROW_BLK, D), lambda i: (i, 0)),
        out_shape=jax.ShapeDtypeStruct((N, D), jnp.float32),
    )(t, partials, partials, wb)


def kernel(x, edge_index, W_msg, b_msg, W_next, b_next):
    src = edge_index[0].astype(jnp.int32)
    dst = edge_index[1].astype(jnp.int32)
    pad = E_PAD - E
    pad_ar = jnp.arange(pad, dtype=jnp.int32)
    idx_shape = (NW, NCHUNKS, CHUNK)
    src_p = jnp.concatenate([src, pad_ar % N]).reshape(idx_shape)
    dst_p = jnp.concatenate([dst, N + pad_ar % (N_PAD - N)])
    dst_p = dst_p.reshape(idx_shape)

    h, t = _msg_transform(x, W_msg, b_msg.reshape(1, D),
                          W_next[:D], b_next.reshape(1, D))
    partials = _make_segment_sum_sc()(h, src_p, dst_p)
    return _next_state(t, partials, W_next[D:])


# speedup vs baseline: 1.0223x; 1.0223x over previous
import functools

import jax
import jax.numpy as jnp
from jax import lax
from jax.experimental import pallas as pl
from jax.experimental.pallas import tpu as pltpu
from jax.experimental.pallas import tpu_sc as plsc

N = 10000
E = 320000
D = 128

NC = 2
NS = 16
NW = NC * NS
CHUNK = 128
NBUF = 2
GROUP = 8
NCHUNKS = 80
NGROUPS = NCHUNKS // GROUP
EPW = NCHUNKS * CHUNK
E_PAD = NW * EPW
N_PAD = 10240
ROWS_PER_TILE = N_PAD // NS
ROW_STEPS = ROWS_PER_TILE // CHUNK

ROW_BLK = 10000
GRID = N // ROW_BLK


def _msg_body(x_ref, w_ref, b_ref, o_ref):
    acc = jnp.dot(x_ref[...], w_ref[...], preferred_element_type=jnp.float32)
    o_ref[...] = jnp.maximum(acc + b_ref[...], 0.0)


def _msg_transform(x, w, b):
    return pl.pallas_call(
        _msg_body,
        grid=(GRID,),
        in_specs=[
            pl.BlockSpec((ROW_BLK, D), lambda i: (i, 0)),
            pl.BlockSpec((D, D), lambda i: (0, 0)),
            pl.BlockSpec((1, D), lambda i: (0, 0)),
        ],
        out_specs=pl.BlockSpec((ROW_BLK, D), lambda i: (i, 0)),
        out_shape=jax.ShapeDtypeStruct((N, D), jnp.float32),
    )(x, w, b)


@functools.cache
def _make_segment_sum_sc():
    mesh = plsc.VectorSubcoreMesh(
        core_axis_name="c", subcore_axis_name="s",
        num_cores=NC, num_subcores=NS)
    return pl.kernel(
        _segment_sum_body,
        out_type=jax.ShapeDtypeStruct((NC, N_PAD, D), jnp.float32),
        mesh=mesh,
        scratch_types=[
            pltpu.VMEM((2, GROUP, CHUNK), jnp.int32),
            pltpu.VMEM((2, GROUP, CHUNK), jnp.int32),
            pltpu.VMEM((NBUF, CHUNK, D), jnp.float32),
            pltpu.VMEM_SHARED((N_PAD, D), jnp.float32),
            [pltpu.SemaphoreType.DMA] * NBUF,
            [pltpu.SemaphoreType.DMA] * NBUF,
        ],
    )


def _segment_sum_body(h_hbm, src_hbm, dst_hbm, out_hbm,
                      src_v, dst_v, rows_v, acc_sh, gsem, ssem):
    c = lax.axis_index("c")
    s = lax.axis_index("s")
    wid = c * NS + s

    pltpu.sync_copy(src_hbm.at[wid, pl.ds(0, GROUP)], src_v.at[0])
    pltpu.sync_copy(dst_hbm.at[wid, pl.ds(0, GROUP)], dst_v.at[0])
    pltpu.async_copy(h_hbm.at[src_v.at[0, 1]], rows_v.at[1], gsem[1])

    zero16 = jnp.zeros((16,), jnp.float32)

    def _z(i, _):
        rows_v[0, i // (D // 16), pl.ds((i % (D // 16)) * 16, 16)] = zero16
        return 0

    lax.fori_loop(0, CHUNK * (D // 16), _z, 0)

    row0 = s * ROWS_PER_TILE

    def _zc(j, _):
        pltpu.sync_copy(rows_v.at[0],
                        acc_sh.at[pl.ds(row0 + j * CHUNK, CHUNK)])
        return 0

    lax.fori_loop(0, ROW_STEPS, _zc, 0)
    plsc.subcore_barrier()
    pltpu.async_copy(h_hbm.at[src_v.at[0, 0]], rows_v.at[0], gsem[0])

    def _outer(g, _):
        p = g % 2
        q = (g + 1) % 2

        @pl.when(g + 1 < NGROUPS)
        def _():
            pltpu.sync_copy(src_hbm.at[wid, pl.ds((g + 1) * GROUP, GROUP)],
                            src_v.at[q])
            pltpu.sync_copy(dst_hbm.at[wid, pl.ds((g + 1) * GROUP, GROUP)],
                            dst_v.at[q])

        for b in range(GROUP):
            r = b % NBUF
            pltpu.make_async_copy(
                h_hbm.at[src_v.at[p, b]], rows_v.at[r], gsem[r]).wait()
            pltpu.async_copy(
                rows_v.at[r], acc_sh.at[dst_v.at[p, b]], ssem[r], add=True)
            pltpu.make_async_copy(
                rows_v.at[r], acc_sh.at[dst_v.at[p, b]], ssem[r]).wait()
            if b + NBUF < GROUP:
                pltpu.async_copy(
                    h_hbm.at[src_v.at[p, b + NBUF]], rows_v.at[r], gsem[r])
            else:
                @pl.when(g + 1 < NGROUPS)
                def _():
                    pltpu.async_copy(
                        h_hbm.at[src_v.at[q, b + NBUF - GROUP]],
                        rows_v.at[r], gsem[r])
        return 0

    lax.fori_loop(0, NGROUPS, _outer, 0)
    plsc.subcore_barrier()

    for j in range(ROW_STEPS):
        r = j % NBUF
        rr = row0 + j * CHUNK
        if j >= NBUF:
            pltpu.make_async_copy(
                rows_v.at[r], out_hbm.at[c, pl.ds(row0, CHUNK)],
                ssem[r]).wait()
        pltpu.sync_copy(acc_sh.at[pl.ds(rr, CHUNK)], rows_v.at[r])
        pltpu.async_copy(rows_v.at[r], out_hbm.at[c, pl.ds(rr, CHUNK)],
                         ssem[r])
    for r in range(min(NBUF, ROW_STEPS)):
        pltpu.make_async_copy(
            rows_v.at[r], out_hbm.at[c, pl.ds(row0, CHUNK)], ssem[r]).wait()


def _self_body(x_ref, wa_ref, b_ref, o_ref):
    o_ref[...] = jnp.dot(x_ref[...], wa_ref[...],
                         preferred_element_type=jnp.float32) + b_ref[...]


def _self_transform(x, wa, b):
    return pl.pallas_call(
        _self_body,
        grid=(GRID,),
        in_specs=[
            pl.BlockSpec((ROW_BLK, D), lambda i: (i, 0)),
            pl.BlockSpec((D, D), lambda i: (0, 0)),
            pl.BlockSpec((1, D), lambda i: (0, 0)),
        ],
        out_specs=pl.BlockSpec((ROW_BLK, D), lambda i: (i, 0)),
        out_shape=jax.ShapeDtypeStruct((N, D), jnp.float32),
    )(x, wa, b)


def _next_body(t_ref, p0_ref, p1_ref, wb_ref, o_ref):
    pooled = p0_ref[0] + p1_ref[0]
    acc = t_ref[...] + jnp.dot(pooled, wb_ref[...],
                               preferred_element_type=jnp.float32)
    o_ref[...] = jnp.maximum(acc, 0.0)


def _next_state(t, partials, wb):
    return pl.pallas_call(
        _next_body,
        grid=(GRID,),
        in_specs=[
            pl.BlockSpec((ROW_BLK, D), lambda i: (i, 0)),
            pl.BlockSpec((1, ROW_BLK, D), lambda i: (0, i, 0)),
            pl.BlockSpec((1, ROW_BLK, D), lambda i: (1, i, 0)),
            pl.BlockSpec((D, D), lambda i: (0, 0)),
        ],
        out_specs=pl.BlockSpec((ROW_BLK, D), lambda i: (i, 0)),
        out_shape=jax.ShapeDtypeStruct((N, D), jnp.float32),
    )(t, partials, partials, wb)


def kernel(x, edge_index, W_msg, b_msg, W_next, b_next):
    src = edge_index[0].astype(jnp.int32)
    dst = edge_index[1].astype(jnp.int32)
    pad = E_PAD - E
    pad_ar = jnp.arange(pad, dtype=jnp.int32)
    idx_shape = (NW, NCHUNKS, CHUNK)
    src_p = jnp.concatenate([src, pad_ar % N]).reshape(idx_shape)
    dst_p = jnp.concatenate([dst, N + pad_ar % (N_PAD - N)])
    dst_p = dst_p.reshape(idx_shape)

    h = _msg_transform(x, W_msg, b_msg.reshape(1, D))
    t = _self_transform(x, W_next[:D], b_next.reshape(1, D))
    partials = _make_segment_sum_sc()(h, src_p, dst_p)
    return _next_state(t, partials, W_next[D:])
